# baseline (speedup 1.0000x reference)
import functools

import jax
import jax.numpy as jnp
from jax import lax
from jax.experimental import pallas as pl
from jax.experimental.pallas import tpu as pltpu
from jax.experimental.pallas import tpu_sc as plsc

N = 10000
E = 320000
D = 128
H = 8
C = 16
ACCW = 144

_NC = 2
_NS = 16
_NW = _NC * _NS
_EPW = E // _NW
_K = 50
_NCHUNK = _EPW // _K
_CPW = _NCHUNK
_RK = 50
_NRCH = N // _RK
_NRITER = (_NRCH + _NS - 1) // _NS

_BLK = 1000


def _head_ind():
    r = lax.broadcasted_iota(jnp.int32, (D, H), 0) // C
    col = lax.broadcasted_iota(jnp.int32, (D, H), 1)
    return (r == col).astype(jnp.float32)


def _attn_tables(xw, a_s, a_d):
    ind = _head_ind()
    asrc8 = jnp.dot(xw * a_s, ind, preferred_element_type=jnp.float32, precision=lax.Precision.HIGHEST)
    adst8 = jnp.dot(xw * a_d, ind, preferred_element_type=jnp.float32, precision=lax.Precision.HIGHEST)
    z = jnp.zeros_like(asrc8)
    return (jnp.concatenate([asrc8, z], axis=1),
            jnp.concatenate([adst8, z], axis=1))


def _mm_bf16(a, b):
    return jnp.dot(a.astype(jnp.bfloat16), b.astype(jnp.bfloat16),
                   preferred_element_type=jnp.float32)


def _prep_first_body(x_ref, W_ref, as_ref, ad_ref, xw_ref, asp_ref, adp_ref):
    xw = _mm_bf16(x_ref[...], W_ref[...])
    xw_ref[...] = xw
    asp, adp = _attn_tables(xw, as_ref[...], ad_ref[...])
    asp_ref[...] = asp
    adp_ref[...] = adp


def _normalized_input(p0_ref, p1_ref, b_ref):
    S = p0_ref[0] + p1_ref[0]
    U = S[:, :D]
    den8 = S[:, D:D + H]
    den = jnp.dot(den8, _head_ind().T, preferred_element_type=jnp.float32, precision=lax.Precision.HIGHEST)
    return jnp.maximum(U / (den + 1e-16) + b_ref[...], 0.0)


def _prep_mid_body(p0_ref, p1_ref, b_ref, W_ref, as_ref, ad_ref,
                   xw_ref, asp_ref, adp_ref):
    xin = _normalized_input(p0_ref, p1_ref, b_ref)
    xw = _mm_bf16(xin, W_ref[...])
    xw_ref[...] = xw
    asp, adp = _attn_tables(xw, as_ref[...], ad_ref[...])
    asp_ref[...] = asp
    adp_ref[...] = adp


def _clf_body(p0_ref, p1_ref, b_ref, Wc1_ref, bc1_ref, Wc2_ref, bc2_ref,
              o_ref):
    xin = _normalized_input(p0_ref, p1_ref, b_ref)
    h1 = jnp.maximum(_mm_bf16(xin, Wc1_ref[...]) + bc1_ref[...], 0.0)
    o_ref[...] = _mm_bf16(h1, Wc2_ref[...]) + bc2_ref[...]


_GRID = N // _BLK

_row_spec = pl.BlockSpec((_BLK, D), lambda i: (i, 0))
_p0_spec = pl.BlockSpec((1, _BLK, ACCW), lambda i: (0, i, 0))
_p1_spec = pl.BlockSpec((1, _BLK, ACCW), lambda i: (1, i, 0))
_w_spec = pl.BlockSpec((D, D), lambda i: (0, 0))
_vec_spec = pl.BlockSpec((1, D), lambda i: (0, 0))
_t16_spec = pl.BlockSpec((_BLK, 2 * H), lambda i: (i, 0))

_prep_out = (jax.ShapeDtypeStruct((N, D), jnp.float32),
             jax.ShapeDtypeStruct((N, 2 * H), jnp.float32),
             jax.ShapeDtypeStruct((N, 2 * H), jnp.float32))
_prep_out_specs = (_row_spec, _t16_spec, _t16_spec)

_prep_first = pl.pallas_call(
    _prep_first_body, grid=(_GRID,),
    in_specs=[_row_spec, _w_spec, _vec_spec, _vec_spec],
    out_specs=_prep_out_specs, out_shape=_prep_out)

_prep_mid = pl.pallas_call(
    _prep_mid_body, grid=(_GRID,),
    in_specs=[_p0_spec, _p1_spec, _vec_spec, _w_spec, _vec_spec, _vec_spec],
    out_specs=_prep_out_specs, out_shape=_prep_out)

_clf = pl.pallas_call(
    _clf_body, grid=(_GRID,),
    in_specs=[_p0_spec, _p1_spec, _vec_spec, _w_spec, _vec_spec,
              pl.BlockSpec((D, 1), lambda i: (0, 0)),
              pl.BlockSpec((1, 1), lambda i: (0, 0))],
    out_specs=pl.BlockSpec((_BLK, 1), lambda i: (i, 0)),
    out_shape=jax.ShapeDtypeStruct((N, 1), jnp.float32))


def _bcast_lane(v, h):
    idx = jnp.full((16, 1), h, dtype=jnp.int32)
    dnums = lax.GatherDimensionNumbers(
        offset_dims=(), collapsed_slice_dims=(0,), start_index_map=(0,))
    return lax.gather(v, idx, dnums, (1,),
                      mode=lax.GatherScatterMode.PROMISE_IN_BOUNDS)


def _sc_edge_kernel(src_hbm, dst_hbm, asrc_hbm, adst_hbm, xw_hbm, out_hbm,
                    sbufs, dbufs, abufs, bbufs, xwbufs, mbufs, acc,
                    semg, sems, semi):
    c = lax.axis_index("c")
    s = lax.axis_index("s")
    wid = c * _NS + s
    row0 = wid * _CPW

    zero = jnp.zeros((16,), jnp.float32)

    @pl.loop(0, _RK)
    def _(i):
        for j in range(ACCW // 16):
            mbufs[0][i, pl.ds(j * 16, 16)] = zero

    @pl.loop(0, _NRITER)
    def _(j):
        t = j * _NS + s

        @pl.when(t < _NRCH)
        def _():
            pltpu.sync_copy(mbufs[0].at[pl.ds(0, _RK)],
                            acc.at[pl.ds(t * _RK, _RK)])

    plsc.subcore_barrier()

    def issue_idx(ci, q, p):
        pltpu.async_copy(src_hbm.at[pl.ds(row0 + ci, 1)],
                         sbufs.at[pl.ds(q, 1)], semi[p])
        pltpu.async_copy(dst_hbm.at[pl.ds(row0 + ci, 1)],
                         dbufs.at[pl.ds(q, 1)], semi[p])

    def wait_idx(q, p):
        pltpu.make_async_copy(src_hbm.at[pl.ds(row0, 1)],
                              sbufs.at[pl.ds(q, 1)], semi[p]).wait()
        pltpu.make_async_copy(dst_hbm.at[pl.ds(row0, 1)],
                              dbufs.at[pl.ds(q, 1)], semi[p]).wait()

    def issue_gathers(q, p):
        pltpu.async_copy(asrc_hbm.at[sbufs.at[q]], abufs[p], semg[p])
        pltpu.async_copy(adst_hbm.at[dbufs.at[q]], bbufs[p], semg[p])
        pltpu.async_copy(xw_hbm.at[sbufs.at[q]], xwbufs[p], semg[p])

    def wait_gathers(q, p):
        pltpu.make_async_copy(asrc_hbm.at[sbufs.at[q]], abufs[p],
                              semg[p]).wait()
        pltpu.make_async_copy(adst_hbm.at[sbufs.at[q]], bbufs[p],
                              semg[p]).wait()
        pltpu.make_async_copy(xw_hbm.at[sbufs.at[q]], xwbufs[p],
                              semg[p]).wait()

    for ci in range(2):
        issue_idx(ci, ci, ci)
        wait_idx(ci, ci)
        issue_gathers(ci, ci)

    @pl.loop(0, _NCHUNK // 4)
    def _(G):
        for gg in range(2):
            for p in range(2):
                q = 2 * gg + p
                ci = G * 4 + q
                wait_gathers(q, p)

                @pl.when(ci > 1)
                def _():
                    pltpu.make_async_copy(mbufs[p], acc.at[dbufs.at[q]],
                                          sems[p]).wait()

                @pl.when(ci < _NCHUNK - 2)
                def _():
                    issue_idx(ci + 2, (q + 2) % 4, p)

                abuf, bbuf, xwbuf, mbuf = (abufs[p], bbufs[p], xwbufs[p],
                                           mbufs[p])

                @plsc.parallel_loop(0, _K, unroll=5)
                def _(e):
                    ev = abuf[e, :] + bbuf[e, :]
                    ev = jnp.maximum(ev, 0.2 * ev)
                    exv = jnp.exp(ev)
                    mbuf[e, pl.ds(D, 16)] = exv
                    for h in range(H):
                        bc = _bcast_lane(exv, h)
                        mbuf[e, pl.ds(h * C, C)] = (
                            xwbuf[e, pl.ds(h * C, C)] * bc)

                pltpu.async_copy(mbuf, acc.at[dbufs.at[q]], sems[p],
                                 add=True)

                @pl.when(ci < _NCHUNK - 2)
                def _():
                    wait_idx((q + 2) % 4, p)
                    issue_gathers((q + 2) % 4, p)

    for p in range(2):
        q = (_NCHUNK - 2 + p) % 4
        pltpu.make_async_copy(mbufs[p], acc.at[dbufs.at[q]], sems[p]).wait()

    plsc.subcore_barrier()

    @pl.loop(0, _NRITER)
    def _(j):
        t = j * _NS + s

        @pl.when(t < _NRCH)
        def _():
            pltpu.sync_copy(acc.at[pl.ds(t * _RK, _RK)],
                            out_hbm.at[c, pl.ds(t * _RK, _RK)])


def _sc_edge_wrapped(src2d, dst2d, asrcP, adstP, xw):
    def body(src_hbm, dst_hbm, asrc_hbm, adst_hbm, xw_hbm, out_hbm,
             sbufs, dbufs, a0, a1, b0, b1, x0, x1, m0, m1, acc,
             sg0, sg1, ss0, ss1, si0, si1):
        _sc_edge_kernel(src_hbm, dst_hbm, asrc_hbm, adst_hbm, xw_hbm,
                        out_hbm, sbufs, dbufs, (a0, a1), (b0, b1), (x0, x1),
                        (m0, m1), acc, (sg0, sg1), (ss0, ss1), (si0, si1))

    f = pl.kernel(
        body,
        out_type=jax.ShapeDtypeStruct((_NC, N, ACCW), jnp.float32),
        mesh=plsc.VectorSubcoreMesh(core_axis_name="c", subcore_axis_name="s"),
        compiler_params=pltpu.CompilerParams(use_tc_tiling_on_sc=False),
        scratch_types=[
            pltpu.VMEM((4, _K), jnp.int32),
            pltpu.VMEM((4, _K), jnp.int32),
            pltpu.VMEM((_K, 2 * H), jnp.float32),
            pltpu.VMEM((_K, 2 * H), jnp.float32),
            pltpu.VMEM((_K, 2 * H), jnp.float32),
            pltpu.VMEM((_K, 2 * H), jnp.float32),
            pltpu.VMEM((_K, D), jnp.float32),
            pltpu.VMEM((_K, D), jnp.float32),
            pltpu.VMEM((_K, ACCW), jnp.float32),
            pltpu.VMEM((_K, ACCW), jnp.float32),
            pltpu.VMEM_SHARED((N, ACCW), jnp.float32),
            pltpu.SemaphoreType.DMA,
            pltpu.SemaphoreType.DMA,
            pltpu.SemaphoreType.DMA,
            pltpu.SemaphoreType.DMA,
            pltpu.SemaphoreType.DMA,
            pltpu.SemaphoreType.DMA,
        ],
    )
    return f(src2d, dst2d, asrcP, adstP, xw)


_sc_edge = _sc_edge_wrapped


def kernel(x, edge_index_dict, W0, a_src0, a_dst0, b0, W1, a_src1, a_dst1,
           b1, W2, a_src2, a_dst2, b2, Wc1, bc1, Wc2, bc2):
    src = edge_index_dict[0].reshape(E // _K, _K)
    dst = edge_index_dict[1].reshape(E // _K, _K)

    xw, asp, adp = _prep_first(x, W0, a_src0.reshape(1, D),
                               a_dst0.reshape(1, D))
    P = _sc_edge(src, dst, asp, adp, xw)
    xw, asp, adp = _prep_mid(P, P, b0.reshape(1, D), W1,
                             a_src1.reshape(1, D), a_dst1.reshape(1, D))
    P = _sc_edge(src, dst, asp, adp, xw)
    xw, asp, adp = _prep_mid(P, P, b1.reshape(1, D), W2,
                             a_src2.reshape(1, D), a_dst2.reshape(1, D))
    P = _sc_edge(src, dst, asp, adp, xw)
    out = _clf(P, P, b2.reshape(1, D), Wc1, bc1.reshape(1, D), Wc2,
               bc2.reshape(1, 1))
    return out[:, 0]


# speedup vs baseline: 1.0023x; 1.0023x over previous
import functools

import jax
import jax.numpy as jnp
from jax import lax
from jax.experimental import pallas as pl
from jax.experimental.pallas import tpu as pltpu
from jax.experimental.pallas import tpu_sc as plsc

N = 10000
E = 320000
D = 128
H = 8
C = 16
ACCW = 144

_NC = 2
_NS = 16
_NW = _NC * _NS
_EPW = E // _NW
_K = 50
_NCHUNK = _EPW // _K
_CPW = _NCHUNK
_RK = 50
_NRCH = N // _RK
_NRITER = (_NRCH + _NS - 1) // _NS

_BLK = 1000


def _head_ind():
    r = lax.broadcasted_iota(jnp.int32, (D, H), 0) // C
    col = lax.broadcasted_iota(jnp.int32, (D, H), 1)
    return (r == col).astype(jnp.float32)


def _attn_tables(xw, a_s, a_d):
    ind = _head_ind()
    asrc8 = jnp.dot(xw * a_s, ind, preferred_element_type=jnp.float32, precision=lax.Precision.HIGHEST)
    adst8 = jnp.dot(xw * a_d, ind, preferred_element_type=jnp.float32, precision=lax.Precision.HIGHEST)
    z = jnp.zeros_like(asrc8)
    return (jnp.concatenate([asrc8, z], axis=1),
            jnp.concatenate([adst8, z], axis=1))


def _mm_bf16(a, b):
    return jnp.dot(a.astype(jnp.bfloat16), b.astype(jnp.bfloat16),
                   preferred_element_type=jnp.float32)


def _prep_first_body(x_ref, W_ref, as_ref, ad_ref, xw_ref, asp_ref, adp_ref):
    xw = _mm_bf16(x_ref[...], W_ref[...])
    xw_ref[...] = xw
    asp, adp = _attn_tables(xw, as_ref[...], ad_ref[...])
    asp_ref[...] = asp
    adp_ref[...] = adp


def _normalized_input(p0_ref, p1_ref, b_ref):
    S = p0_ref[0] + p1_ref[0]
    U = S[:, :D]
    den8 = S[:, D:D + H]
    den = jnp.dot(den8, _head_ind().T, preferred_element_type=jnp.float32, precision=lax.Precision.HIGHEST)
    return jnp.maximum(U / (den + 1e-16) + b_ref[...], 0.0)


def _prep_mid_body(p0_ref, p1_ref, b_ref, W_ref, as_ref, ad_ref,
                   xw_ref, asp_ref, adp_ref):
    xin = _normalized_input(p0_ref, p1_ref, b_ref)
    xw = _mm_bf16(xin, W_ref[...])
    xw_ref[...] = xw
    asp, adp = _attn_tables(xw, as_ref[...], ad_ref[...])
    asp_ref[...] = asp
    adp_ref[...] = adp


def _clf_body(p0_ref, p1_ref, b_ref, Wc1_ref, bc1_ref, Wc2_ref, bc2_ref,
              o_ref):
    xin = _normalized_input(p0_ref, p1_ref, b_ref)
    h1 = jnp.maximum(_mm_bf16(xin, Wc1_ref[...]) + bc1_ref[...], 0.0)
    o_ref[...] = _mm_bf16(h1, Wc2_ref[...]) + bc2_ref[...]


_GRID = N // _BLK

_row_spec = pl.BlockSpec((_BLK, D), lambda i: (i, 0))
_p0_spec = pl.BlockSpec((1, _BLK, ACCW), lambda i: (0, i, 0))
_p1_spec = pl.BlockSpec((1, _BLK, ACCW), lambda i: (1, i, 0))
_w_spec = pl.BlockSpec((D, D), lambda i: (0, 0))
_vec_spec = pl.BlockSpec((1, D), lambda i: (0, 0))
_t16_spec = pl.BlockSpec((_BLK, 2 * H), lambda i: (i, 0))

_prep_out = (jax.ShapeDtypeStruct((N, D), jnp.float32),
             jax.ShapeDtypeStruct((N, 2 * H), jnp.float32),
             jax.ShapeDtypeStruct((N, 2 * H), jnp.float32))
_prep_out_specs = (_row_spec, _t16_spec, _t16_spec)

_prep_first = pl.pallas_call(
    _prep_first_body, grid=(_GRID,),
    in_specs=[_row_spec, _w_spec, _vec_spec, _vec_spec],
    out_specs=_prep_out_specs, out_shape=_prep_out)

_prep_mid = pl.pallas_call(
    _prep_mid_body, grid=(_GRID,),
    in_specs=[_p0_spec, _p1_spec, _vec_spec, _w_spec, _vec_spec, _vec_spec],
    out_specs=_prep_out_specs, out_shape=_prep_out)

_clf = pl.pallas_call(
    _clf_body, grid=(_GRID,),
    in_specs=[_p0_spec, _p1_spec, _vec_spec, _w_spec, _vec_spec,
              pl.BlockSpec((D, 1), lambda i: (0, 0)),
              pl.BlockSpec((1, 1), lambda i: (0, 0))],
    out_specs=pl.BlockSpec((_BLK, 1), lambda i: (i, 0)),
    out_shape=jax.ShapeDtypeStruct((N, 1), jnp.float32))


def _bcast_lane(v, h):
    idx = jnp.full((16, 1), h, dtype=jnp.int32)
    dnums = lax.GatherDimensionNumbers(
        offset_dims=(), collapsed_slice_dims=(0,), start_index_map=(0,))
    return lax.gather(v, idx, dnums, (1,),
                      mode=lax.GatherScatterMode.PROMISE_IN_BOUNDS)


def _sc_edge_kernel(src_hbm, dst_hbm, asrc_hbm, adst_hbm, xw_hbm, out_hbm,
                    sbufs, dbufs, abufs, bbufs, xwbufs, mbufs, acc,
                    semg, sems, semi):
    c = lax.axis_index("c")
    s = lax.axis_index("s")
    wid = c * _NS + s
    row0 = wid * _CPW

    zero = jnp.zeros((16,), jnp.float32)

    @pl.loop(0, _RK)
    def _(i):
        for j in range(ACCW // 16):
            mbufs[0][i, pl.ds(j * 16, 16)] = zero

    @pl.loop(0, _NRITER)
    def _(j):
        t = j * _NS + s

        @pl.when(t < _NRCH)
        def _():
            pltpu.sync_copy(mbufs[0].at[pl.ds(0, _RK)],
                            acc.at[pl.ds(t * _RK, _RK)])

    plsc.subcore_barrier()

    def issue_idx(ci, q, p):
        pltpu.async_copy(src_hbm.at[pl.ds(row0 + ci, 1)],
                         sbufs.at[pl.ds(q, 1)], semi[p])
        pltpu.async_copy(dst_hbm.at[pl.ds(row0 + ci, 1)],
                         dbufs.at[pl.ds(q, 1)], semi[p])

    def wait_idx(q, p):
        pltpu.make_async_copy(src_hbm.at[pl.ds(row0, 1)],
                              sbufs.at[pl.ds(q, 1)], semi[p]).wait()
        pltpu.make_async_copy(dst_hbm.at[pl.ds(row0, 1)],
                              dbufs.at[pl.ds(q, 1)], semi[p]).wait()

    def issue_gathers(q, p):
        pltpu.async_copy(asrc_hbm.at[sbufs.at[q]], abufs[p], semg[p])
        pltpu.async_copy(adst_hbm.at[dbufs.at[q]], bbufs[p], semg[p])
        pltpu.async_copy(xw_hbm.at[sbufs.at[q]], xwbufs[p], semg[p])

    def wait_gathers(q, p):
        pltpu.make_async_copy(asrc_hbm.at[sbufs.at[q]], abufs[p],
                              semg[p]).wait()
        pltpu.make_async_copy(adst_hbm.at[sbufs.at[q]], bbufs[p],
                              semg[p]).wait()
        pltpu.make_async_copy(xw_hbm.at[sbufs.at[q]], xwbufs[p],
                              semg[p]).wait()

    for ci in range(2):
        issue_idx(ci, ci, ci)
        wait_idx(ci, ci)
        issue_gathers(ci, ci)

    @pl.loop(0, _NCHUNK // 4)
    def _(G):
        for gg in range(2):
            for p in range(2):
                q = 2 * gg + p
                ci = G * 4 + q
                wait_gathers(q, p)

                @pl.when(ci > 1)
                def _():
                    pltpu.make_async_copy(mbufs[p], acc.at[dbufs.at[q]],
                                          sems[p]).wait()

                @pl.when(ci < _NCHUNK - 2)
                def _():
                    issue_idx(ci + 2, (q + 2) % 4, p)

                abuf, bbuf, xwbuf, mbuf = (abufs[p], bbufs[p], xwbufs[p],
                                           mbufs[p])

                @plsc.parallel_loop(0, _K, unroll=2)
                def _(e):
                    ev = abuf[e, :] + bbuf[e, :]
                    ev = jnp.maximum(ev, 0.2 * ev)
                    exv = jnp.exp(ev)
                    mbuf[e, pl.ds(D, 16)] = exv
                    for h in range(H):
                        bc = _bcast_lane(exv, h)
                        mbuf[e, pl.ds(h * C, C)] = (
                            xwbuf[e, pl.ds(h * C, C)] * bc)

                pltpu.async_copy(mbuf, acc.at[dbufs.at[q]], sems[p],
                                 add=True)

                @pl.when(ci < _NCHUNK - 2)
                def _():
                    wait_idx((q + 2) % 4, p)
                    issue_gathers((q + 2) % 4, p)

    for p in range(2):
        q = (_NCHUNK - 2 + p) % 4
        pltpu.make_async_copy(mbufs[p], acc.at[dbufs.at[q]], sems[p]).wait()

    plsc.subcore_barrier()

    @pl.loop(0, _NRITER)
    def _(j):
        t = j * _NS + s

        @pl.when(t < _NRCH)
        def _():
            pltpu.sync_copy(acc.at[pl.ds(t * _RK, _RK)],
                            out_hbm.at[c, pl.ds(t * _RK, _RK)])


def _sc_edge_wrapped(src2d, dst2d, asrcP, adstP, xw):
    def body(src_hbm, dst_hbm, asrc_hbm, adst_hbm, xw_hbm, out_hbm,
             sbufs, dbufs, a0, a1, b0, b1, x0, x1, m0, m1, acc,
             sg0, sg1, ss0, ss1, si0, si1):
        _sc_edge_kernel(src_hbm, dst_hbm, asrc_hbm, adst_hbm, xw_hbm,
                        out_hbm, sbufs, dbufs, (a0, a1), (b0, b1), (x0, x1),
                        (m0, m1), acc, (sg0, sg1), (ss0, ss1), (si0, si1))

    f = pl.kernel(
        body,
        out_type=jax.ShapeDtypeStruct((_NC, N, ACCW), jnp.float32),
        mesh=plsc.VectorSubcoreMesh(core_axis_name="c", subcore_axis_name="s"),
        compiler_params=pltpu.CompilerParams(use_tc_tiling_on_sc=False),
        scratch_types=[
            pltpu.VMEM((4, _K), jnp.int32),
            pltpu.VMEM((4, _K), jnp.int32),
            pltpu.VMEM((_K, 2 * H), jnp.float32),
            pltpu.VMEM((_K, 2 * H), jnp.float32),
            pltpu.VMEM((_K, 2 * H), jnp.float32),
            pltpu.VMEM((_K, 2 * H), jnp.float32),
            pltpu.VMEM((_K, D), jnp.float32),
            pltpu.VMEM((_K, D), jnp.float32),
            pltpu.VMEM((_K, ACCW), jnp.float32),
            pltpu.VMEM((_K, ACCW), jnp.float32),
            pltpu.VMEM_SHARED((N, ACCW), jnp.float32),
            pltpu.SemaphoreType.DMA,
            pltpu.SemaphoreType.DMA,
            pltpu.SemaphoreType.DMA,
            pltpu.SemaphoreType.DMA,
            pltpu.SemaphoreType.DMA,
            pltpu.SemaphoreType.DMA,
        ],
    )
    return f(src2d, dst2d, asrcP, adstP, xw)


_sc_edge = _sc_edge_wrapped


def kernel(x, edge_index_dict, W0, a_src0, a_dst0, b0, W1, a_src1, a_dst1,
           b1, W2, a_src2, a_dst2, b2, Wc1, bc1, Wc2, bc2):
    src = edge_index_dict[0].reshape(E // _K, _K)
    dst = edge_index_dict[1].reshape(E // _K, _K)

    xw, asp, adp = _prep_first(x, W0, a_src0.reshape(1, D),
                               a_dst0.reshape(1, D))
    P = _sc_edge(src, dst, asp, adp, xw)
    xw, asp, adp = _prep_mid(P, P, b0.reshape(1, D), W1,
                             a_src1.reshape(1, D), a_dst1.reshape(1, D))
    P = _sc_edge(src, dst, asp, adp, xw)
    xw, asp, adp = _prep_mid(P, P, b1.reshape(1, D), W2,
                             a_src2.reshape(1, D), a_dst2.reshape(1, D))
    P = _sc_edge(src, dst, asp, adp, xw)
    out = _clf(P, P, b2.reshape(1, D), Wc1, bc1.reshape(1, D), Wc2,
               bc2.reshape(1, 1))
    return out[:, 0]
